# initial kernel scaffold (unmeasured)
import jax
import jax.numpy as jnp
from jax import lax
from jax.experimental import pallas as pl
from jax.experimental.pallas import tpu as pltpu

N_DEV = 32


def kernel(x, router_W, route_idx, expert_W):
    n_tok, d_model = x.shape
    e_local, _, d_ff = expert_W.shape

    def body(x_ref, rw_ref, idx_ref, ew_ref, out_ref,
             comm_ref, send_sems, recv_sems, ack_sem):
        me = lax.axis_index("i")
        left = lax.rem(me + N_DEV - 1, N_DEV)
        right = lax.rem(me + 1, N_DEV)

        barrier = pltpu.get_barrier_semaphore()
        pl.semaphore_signal(barrier, inc=1, device_id=(left,),
                            device_id_type=pl.DeviceIdType.MESH)
        pl.semaphore_signal(barrier, inc=1, device_id=(right,),
                            device_id_type=pl.DeviceIdType.MESH)
        pl.semaphore_wait(barrier, 2)

        xv = x_ref[...]
        scores = jnp.dot(xv, rw_ref[...], preferred_element_type=jnp.float32)
        smax = jnp.max(scores, axis=-1, keepdims=True)
        probs = jnp.exp(scores - smax)
        probs = probs / jnp.sum(probs, axis=-1, keepdims=True)
        eids = lax.broadcasted_iota(jnp.int32, scores.shape, 1)
        r0 = idx_ref[:, 0:1]
        r1 = idx_ref[:, 1:2]
        p0 = jnp.sum(jnp.where(eids == r0, probs, 0.0), axis=-1, keepdims=True)
        p1 = jnp.sum(jnp.where(eids == r1, probs, 0.0), axis=-1, keepdims=True)
        gsum = p0 + p1
        g0 = p0 / gsum
        g1 = p1 / gsum

        out_ref[...] = jnp.zeros((n_tok, d_ff), jnp.float32)
        comm_ref[0] = ew_ref[...]

        def do_hop(h, slot):
            nslot = 1 - slot
            src = lax.rem(me + N_DEV - h, N_DEV)
            rdma = pltpu.make_async_remote_copy(
                src_ref=comm_ref.at[slot],
                dst_ref=comm_ref.at[nslot],
                send_sem=send_sems.at[slot],
                recv_sem=recv_sems.at[nslot],
                device_id=(right,),
                device_id_type=pl.DeviceIdType.MESH,
            )

            @pl.when(jnp.logical_and(h >= 1, h < N_DEV - 1))
            def _():
                pl.semaphore_wait(ack_sem, 1)

            @pl.when(h < N_DEV - 1)
            def _():
                rdma.start()

            acc = jnp.zeros((n_tok, d_ff), jnp.float32)
            for j in range(e_local):
                e = e_local * src + j
                g = (jnp.where(r0 == e, g0, 0.0)
                     + jnp.where(r1 == e, g1, 0.0))
                acc = acc + jnp.dot(xv * g, comm_ref[slot, j],
                                    preferred_element_type=jnp.float32)
            out_ref[...] = out_ref[...] + acc

            @pl.when(h < N_DEV - 1)
            def _():
                rdma.wait()

            @pl.when(h < N_DEV - 2)
            def _():
                pl.semaphore_signal(ack_sem, inc=1, device_id=(left,),
                                    device_id_type=pl.DeviceIdType.MESH)

        def pair(k, carry):
            do_hop(2 * k, 0)
            do_hop(2 * k + 1, 1)
            return carry

        lax.fori_loop(0, N_DEV // 2, pair, 0)

    return pl.pallas_call(
        body,
        out_shape=jax.ShapeDtypeStruct((n_tok, d_ff), jnp.float32),
        in_specs=[pl.BlockSpec(memory_space=pltpu.VMEM)] * 4,
        out_specs=pl.BlockSpec(memory_space=pltpu.VMEM),
        scratch_shapes=[
            pltpu.VMEM((2, e_local, d_model, d_ff), jnp.float32),
            pltpu.SemaphoreType.DMA((2,)),
            pltpu.SemaphoreType.DMA((2,)),
            pltpu.SemaphoreType.REGULAR,
        ],
        compiler_params=pltpu.CompilerParams(collective_id=0),
    )(x, router_W, route_idx, expert_W)


# baseline (device time: 3040677 ns/iter reference)
import jax
import jax.numpy as jnp
from jax import lax
from jax.experimental import pallas as pl
from jax.experimental.pallas import tpu as pltpu

N_DEV = 32


def kernel(x, router_W, route_idx, expert_W):
    n_tok, d_model = x.shape
    e_local, _, d_ff = expert_W.shape

    def body(x_ref, rw_ref, idx_ref, ew_ref, out_ref,
             comm_ref, send_sems, recv_sems, ack_sem):
        me = lax.axis_index("i")
        left = lax.rem(me + N_DEV - 1, N_DEV)
        right = lax.rem(me + 1, N_DEV)

        barrier = pltpu.get_barrier_semaphore()
        pl.semaphore_signal(barrier, inc=1, device_id=(left,),
                            device_id_type=pl.DeviceIdType.MESH)
        pl.semaphore_signal(barrier, inc=1, device_id=(right,),
                            device_id_type=pl.DeviceIdType.MESH)
        pl.semaphore_wait(barrier, 2)

        xv = x_ref[...]
        scores = jnp.dot(xv, rw_ref[...], preferred_element_type=jnp.float32)
        smax = jnp.max(scores, axis=-1, keepdims=True)
        probs = jnp.exp(scores - smax)
        probs = probs / jnp.sum(probs, axis=-1, keepdims=True)
        eids = lax.broadcasted_iota(jnp.int32, scores.shape, 1)
        r0 = idx_ref[:, 0:1]
        r1 = idx_ref[:, 1:2]
        p0 = jnp.sum(jnp.where(eids == r0, probs, 0.0), axis=-1, keepdims=True)
        p1 = jnp.sum(jnp.where(eids == r1, probs, 0.0), axis=-1, keepdims=True)
        gsum = p0 + p1
        g0 = p0 / gsum
        g1 = p1 / gsum

        out_ref[...] = jnp.zeros((n_tok, d_ff), jnp.float32)
        comm_ref[0] = ew_ref[...]

        def do_hop(h, slot):
            nslot = 1 - slot
            src = lax.rem(me + N_DEV - h, N_DEV)
            rdma = pltpu.make_async_remote_copy(
                src_ref=comm_ref.at[slot],
                dst_ref=comm_ref.at[nslot],
                send_sem=send_sems.at[slot],
                recv_sem=recv_sems.at[nslot],
                device_id=(right,),
                device_id_type=pl.DeviceIdType.MESH,
            )

            @pl.when(jnp.logical_and(h >= 1, h < N_DEV - 1))
            def _():
                pl.semaphore_wait(ack_sem, 1)

            @pl.when(h < N_DEV - 1)
            def _():
                rdma.start()

            acc = jnp.zeros((n_tok, d_ff), jnp.float32)
            for j in range(e_local):
                e = e_local * src + j
                g = (jnp.where(r0 == e, g0, 0.0)
                     + jnp.where(r1 == e, g1, 0.0))
                acc = acc + jnp.dot(xv * g, comm_ref[slot, j],
                                    preferred_element_type=jnp.float32)
            out_ref[...] = out_ref[...] + acc

            @pl.when(h < N_DEV - 1)
            def _():
                rdma.wait()

            @pl.when(h < N_DEV - 2)
            def _():
                pl.semaphore_signal(ack_sem, inc=1, device_id=(left,),
                                    device_id_type=pl.DeviceIdType.MESH)

        def pair(k, carry):
            do_hop(2 * k, 0)
            do_hop(2 * k + 1, 1)
            return carry

        lax.fori_loop(0, N_DEV // 2, pair, 0)

    return pl.pallas_call(
        body,
        out_shape=jax.ShapeDtypeStruct((n_tok, d_ff), jnp.float32),
        in_specs=[pl.BlockSpec(memory_space=pltpu.VMEM)] * 4,
        out_specs=pl.BlockSpec(memory_space=pltpu.VMEM),
        scratch_shapes=[
            pltpu.VMEM((2, e_local, d_model, d_ff), jnp.float32),
            pltpu.SemaphoreType.DMA((2,)),
            pltpu.SemaphoreType.DMA((2,)),
            pltpu.SemaphoreType.REGULAR,
        ],
        compiler_params=pltpu.CompilerParams(
            collective_id=0,
            vmem_limit_bytes=100 * 1024 * 1024,
        ),
    )(x, router_W, route_idx, expert_W)


# device time: 1653340 ns/iter; 1.8391x vs baseline; 1.8391x over previous
import jax
import jax.numpy as jnp
from jax import lax
from jax.experimental import pallas as pl
from jax.experimental.pallas import tpu as pltpu

N_DEV = 32
CAP = 64


def _moe_ring(xg, expert_W):
    n_rows, d_model = xg.shape
    e_local, _, d_ff = expert_W.shape

    def body(xg_ref, ew_ref, y_ref,
             comm_cw, comm_ccw, send_cw, recv_cw, send_ccw, recv_ccw,
             ack_cw, ack_ccw):
        me = lax.axis_index("i")
        left = lax.rem(me + N_DEV - 1, N_DEV)
        right = lax.rem(me + 1, N_DEV)

        barrier = pltpu.get_barrier_semaphore()
        pl.semaphore_signal(barrier, inc=1, device_id=(left,),
                            device_id_type=pl.DeviceIdType.MESH)
        pl.semaphore_signal(barrier, inc=1, device_id=(right,),
                            device_id_type=pl.DeviceIdType.MESH)
        pl.semaphore_wait(barrier, 2)

        comm_cw[0] = ew_ref[0:2]
        comm_ccw[0] = ew_ref[2:4]

        def do_hop(h, slot):
            nslot = 1 - slot
            src_cw = lax.rem(me + N_DEV - h, N_DEV)
            src_ccw = lax.rem(me + h, N_DEV)
            rdma_cw = pltpu.make_async_remote_copy(
                src_ref=comm_cw.at[slot], dst_ref=comm_cw.at[nslot],
                send_sem=send_cw.at[slot], recv_sem=recv_cw.at[nslot],
                device_id=(right,), device_id_type=pl.DeviceIdType.MESH,
            )
            rdma_ccw = pltpu.make_async_remote_copy(
                src_ref=comm_ccw.at[slot], dst_ref=comm_ccw.at[nslot],
                send_sem=send_ccw.at[slot], recv_sem=recv_ccw.at[nslot],
                device_id=(left,), device_id_type=pl.DeviceIdType.MESH,
            )

            @pl.when(jnp.logical_and(h >= 1, h < N_DEV - 1))
            def _():
                pl.semaphore_wait(ack_cw, 1)
                pl.semaphore_wait(ack_ccw, 1)

            @pl.when(h < N_DEV - 1)
            def _():
                rdma_cw.start()
                rdma_ccw.start()

            for j in range(2):
                e = e_local * src_cw + j
                y_ref[pl.ds(e * CAP, CAP), :] = jnp.dot(
                    xg_ref[pl.ds(e * CAP, CAP), :], comm_cw[slot, j],
                    preferred_element_type=jnp.float32)
            for j in range(2):
                e = e_local * src_ccw + 2 + j
                y_ref[pl.ds(e * CAP, CAP), :] = jnp.dot(
                    xg_ref[pl.ds(e * CAP, CAP), :], comm_ccw[slot, j],
                    preferred_element_type=jnp.float32)

            @pl.when(h < N_DEV - 1)
            def _():
                rdma_cw.wait()
                rdma_ccw.wait()

            @pl.when(h < N_DEV - 2)
            def _():
                pl.semaphore_signal(ack_cw, inc=1, device_id=(left,),
                                    device_id_type=pl.DeviceIdType.MESH)
                pl.semaphore_signal(ack_ccw, inc=1, device_id=(right,),
                                    device_id_type=pl.DeviceIdType.MESH)

        def pair(k, carry):
            do_hop(2 * k, 0)
            do_hop(2 * k + 1, 1)
            return carry

        lax.fori_loop(0, N_DEV // 2, pair, 0)

    return pl.pallas_call(
        body,
        out_shape=jax.ShapeDtypeStruct((n_rows, d_ff), jnp.float32),
        in_specs=[pl.BlockSpec(memory_space=pltpu.VMEM)] * 2,
        out_specs=pl.BlockSpec(memory_space=pltpu.VMEM),
        scratch_shapes=[
            pltpu.VMEM((2, 2, d_model, d_ff), jnp.bfloat16),
            pltpu.VMEM((2, 2, d_model, d_ff), jnp.bfloat16),
            pltpu.SemaphoreType.DMA((2,)),
            pltpu.SemaphoreType.DMA((2,)),
            pltpu.SemaphoreType.DMA((2,)),
            pltpu.SemaphoreType.DMA((2,)),
            pltpu.SemaphoreType.REGULAR,
            pltpu.SemaphoreType.REGULAR,
        ],
        compiler_params=pltpu.CompilerParams(
            collective_id=0,
            vmem_limit_bytes=100 * 1024 * 1024,
        ),
    )(xg, expert_W)


def kernel(x, router_W, route_idx, expert_W):
    n_tok, d_model = x.shape
    n_exp = router_W.shape[1]
    d_ff = expert_W.shape[2]
    n_pairs = 2 * n_tok

    scores = x @ router_W
    probs = jax.nn.softmax(scores, axis=-1)
    p = jnp.take_along_axis(probs, route_idx, axis=1)
    g = p / jnp.sum(p, axis=1, keepdims=True)

    pair_e = route_idx.reshape(-1)
    pair_tok = jnp.repeat(jnp.arange(n_tok, dtype=jnp.int32), 2)
    pair_g = g.reshape(-1)
    order = jnp.argsort(pair_e)
    se = pair_e[order]
    st = pair_tok[order]
    sg = pair_g[order]
    idx = jnp.arange(n_pairs, dtype=jnp.int32)
    is_start = jnp.concatenate([jnp.ones((1,), bool), se[1:] != se[:-1]])
    seg_start = lax.cummax(jnp.where(is_start, idx, 0))
    rank = idx - seg_start
    slot = jnp.where(rank < CAP, se * CAP + rank, n_exp * CAP)

    xg = jnp.zeros((n_exp * CAP, d_model), jnp.bfloat16)
    xg = xg.at[slot].set((x[st] * sg[:, None]).astype(jnp.bfloat16),
                         mode="drop")
    tok_of_slot = jnp.full((n_exp * CAP,), n_tok, jnp.int32)
    tok_of_slot = tok_of_slot.at[slot].set(st, mode="drop")

    y = _moe_ring(xg, expert_W.astype(jnp.bfloat16))

    out = jnp.zeros((n_tok, d_ff), jnp.float32)
    return out.at[tok_of_slot].add(y, mode="drop")


# device time: 1620407 ns/iter; 1.8765x vs baseline; 1.0203x over previous
import jax
import jax.numpy as jnp
from jax import lax
from jax.experimental import pallas as pl
from jax.experimental.pallas import tpu as pltpu

N_DEV = 32
CAP = 64


def _moe_ring(xg, expert_W):
    n_rows, d_model = xg.shape
    e_local, _, d_ff = expert_W.shape

    def body(xg_ref, ew_ref, y_ref,
             comm_cw, comm_ccw, send_cw, recv_cw, send_ccw, recv_ccw,
             ack_cw, ack_ccw):
        me = lax.axis_index("i")
        left = lax.rem(me + N_DEV - 1, N_DEV)
        right = lax.rem(me + 1, N_DEV)

        barrier = pltpu.get_barrier_semaphore()
        pl.semaphore_signal(barrier, inc=1, device_id=(left,),
                            device_id_type=pl.DeviceIdType.MESH)
        pl.semaphore_signal(barrier, inc=1, device_id=(right,),
                            device_id_type=pl.DeviceIdType.MESH)
        pl.semaphore_wait(barrier, 2)

        comm_cw[0] = ew_ref[0:2]
        comm_ccw[0] = ew_ref[2:4]

        def do_hop(h, slot):
            nslot = 1 - slot
            src_cw = lax.rem(me + N_DEV - h, N_DEV)
            src_ccw = lax.rem(me + h, N_DEV)
            rdma_cw = pltpu.make_async_remote_copy(
                src_ref=comm_cw.at[slot], dst_ref=comm_cw.at[nslot],
                send_sem=send_cw.at[slot], recv_sem=recv_cw.at[nslot],
                device_id=(right,), device_id_type=pl.DeviceIdType.MESH,
            )
            rdma_ccw = pltpu.make_async_remote_copy(
                src_ref=comm_ccw.at[slot], dst_ref=comm_ccw.at[nslot],
                send_sem=send_ccw.at[slot], recv_sem=recv_ccw.at[nslot],
                device_id=(left,), device_id_type=pl.DeviceIdType.MESH,
            )

            @pl.when(jnp.logical_and(h >= 1, h < N_DEV - 1))
            def _():
                pl.semaphore_wait(ack_cw, 1)
                pl.semaphore_wait(ack_ccw, 1)

            @pl.when(h < N_DEV - 1)
            def _():
                rdma_cw.start()
                rdma_ccw.start()

            for j in range(2):
                e = e_local * src_cw + j
                y_ref[pl.ds(e * CAP, CAP), :] = jnp.dot(
                    xg_ref[pl.ds(e * CAP, CAP), :], comm_cw[slot, j],
                    preferred_element_type=jnp.float32,
                ).astype(jnp.bfloat16)
            for j in range(2):
                e = e_local * src_ccw + 2 + j
                y_ref[pl.ds(e * CAP, CAP), :] = jnp.dot(
                    xg_ref[pl.ds(e * CAP, CAP), :], comm_ccw[slot, j],
                    preferred_element_type=jnp.float32,
                ).astype(jnp.bfloat16)

            @pl.when(h < N_DEV - 1)
            def _():
                rdma_cw.wait()
                rdma_ccw.wait()

            @pl.when(h < N_DEV - 2)
            def _():
                pl.semaphore_signal(ack_cw, inc=1, device_id=(left,),
                                    device_id_type=pl.DeviceIdType.MESH)
                pl.semaphore_signal(ack_ccw, inc=1, device_id=(right,),
                                    device_id_type=pl.DeviceIdType.MESH)

        def pair(k, carry):
            do_hop(2 * k, 0)
            do_hop(2 * k + 1, 1)
            return carry

        lax.fori_loop(0, N_DEV // 2, pair, 0)

    return pl.pallas_call(
        body,
        out_shape=jax.ShapeDtypeStruct((n_rows, d_ff), jnp.bfloat16),
        in_specs=[pl.BlockSpec(memory_space=pltpu.VMEM)] * 2,
        out_specs=pl.BlockSpec(memory_space=pltpu.VMEM),
        scratch_shapes=[
            pltpu.VMEM((2, 2, d_model, d_ff), jnp.bfloat16),
            pltpu.VMEM((2, 2, d_model, d_ff), jnp.bfloat16),
            pltpu.SemaphoreType.DMA((2,)),
            pltpu.SemaphoreType.DMA((2,)),
            pltpu.SemaphoreType.DMA((2,)),
            pltpu.SemaphoreType.DMA((2,)),
            pltpu.SemaphoreType.REGULAR,
            pltpu.SemaphoreType.REGULAR,
        ],
        compiler_params=pltpu.CompilerParams(
            collective_id=0,
            vmem_limit_bytes=100 * 1024 * 1024,
        ),
    )(xg, expert_W)


def kernel(x, router_W, route_idx, expert_W):
    n_tok, d_model = x.shape
    n_exp = router_W.shape[1]
    d_ff = expert_W.shape[2]
    n_pairs = 2 * n_tok

    scores = x @ router_W
    probs = jax.nn.softmax(scores, axis=-1)
    p = jnp.take_along_axis(probs, route_idx, axis=1)
    g = p / jnp.sum(p, axis=1, keepdims=True)

    pair_e = route_idx.reshape(-1)
    pair_tok = jnp.repeat(jnp.arange(n_tok, dtype=jnp.int32), 2)
    pair_g = g.reshape(-1)
    order = jnp.argsort(pair_e)
    se = pair_e[order]
    st = pair_tok[order]
    sg = pair_g[order]
    idx = jnp.arange(n_pairs, dtype=jnp.int32)
    is_start = jnp.concatenate([jnp.ones((1,), bool), se[1:] != se[:-1]])
    seg_start = lax.cummax(jnp.where(is_start, idx, 0))
    rank = idx - seg_start
    slot = jnp.where(rank < CAP, se * CAP + rank, n_exp * CAP)

    xg = jnp.zeros((n_exp * CAP, d_model), jnp.bfloat16)
    xg = xg.at[slot].set((x[st] * sg[:, None]).astype(jnp.bfloat16),
                         mode="drop")

    y = _moe_ring(xg, expert_W.astype(jnp.bfloat16))

    pair_slot = jnp.zeros((n_pairs,), jnp.int32).at[order].set(slot)
    y0 = jnp.take(y, pair_slot[0::2], axis=0, mode="fill", fill_value=0)
    y1 = jnp.take(y, pair_slot[1::2], axis=0, mode="fill", fill_value=0)
    return y0.astype(jnp.float32) + y1.astype(jnp.float32)


# device time: 973370 ns/iter; 3.1239x vs baseline; 1.6647x over previous
import numpy as np
import jax
import jax.numpy as jnp
from jax import lax
from jax.experimental import pallas as pl
from jax.experimental.pallas import tpu as pltpu

N_DEV = 32
CAP = 64

_CYCLE = np.array(
    [0, 3, 4, 7, 15, 12, 11, 8, 16, 19, 20, 23, 31, 28, 27, 24,
     25, 26, 29, 30, 22, 21, 18, 17, 9, 10, 13, 14, 6, 5, 2, 1],
    dtype=np.int32)
_POS = np.empty(N_DEV, dtype=np.int32)
_POS[_CYCLE] = np.arange(N_DEV, dtype=np.int32)
_NEXT = np.empty(N_DEV, dtype=np.int32)
_NEXT[_CYCLE] = _CYCLE[(np.arange(N_DEV) + 1) % N_DEV]
_PREV = np.empty(N_DEV, dtype=np.int32)
_PREV[_CYCLE] = _CYCLE[(np.arange(N_DEV) - 1) % N_DEV]


def _moe_ring(xg, expert_W, left, right):
    n_rows, d_model = xg.shape
    e_local, _, d_ff = expert_W.shape
    half = (n_rows // 2)

    def body(xg_ref, ew_ref, l_ref, r_ref, y_ref,
             comm_cw, comm_ccw, send_cw, recv_cw, send_ccw, recv_ccw,
             ack_cw, ack_ccw):
        lft = l_ref[0]
        rgt = r_ref[0]

        barrier = pltpu.get_barrier_semaphore()
        pl.semaphore_signal(barrier, inc=1, device_id=(lft,),
                            device_id_type=pl.DeviceIdType.MESH)
        pl.semaphore_signal(barrier, inc=1, device_id=(rgt,),
                            device_id_type=pl.DeviceIdType.MESH)
        pl.semaphore_wait(barrier, 2)

        comm_cw[0] = ew_ref[0:2]
        comm_ccw[0] = ew_ref[2:4]

        def do_hop(h, slot):
            nslot = 1 - slot
            rdma_cw = pltpu.make_async_remote_copy(
                src_ref=comm_cw.at[slot], dst_ref=comm_cw.at[nslot],
                send_sem=send_cw.at[slot], recv_sem=recv_cw.at[nslot],
                device_id=(rgt,), device_id_type=pl.DeviceIdType.MESH,
            )
            rdma_ccw = pltpu.make_async_remote_copy(
                src_ref=comm_ccw.at[slot], dst_ref=comm_ccw.at[nslot],
                send_sem=send_ccw.at[slot], recv_sem=recv_ccw.at[nslot],
                device_id=(lft,), device_id_type=pl.DeviceIdType.MESH,
            )

            @pl.when(jnp.logical_and(h >= 1, h < N_DEV - 1))
            def _():
                pl.semaphore_wait(ack_cw, 1)
                pl.semaphore_wait(ack_ccw, 1)

            @pl.when(h < N_DEV - 1)
            def _():
                rdma_cw.start()
                rdma_ccw.start()

            for j in range(2):
                row = (2 * h + j) * CAP
                y_ref[pl.ds(row, CAP), :] = jnp.dot(
                    xg_ref[pl.ds(row, CAP), :], comm_cw[slot, j],
                    preferred_element_type=jnp.float32,
                ).astype(jnp.bfloat16)
            for j in range(2):
                row = half + (2 * h + j) * CAP
                y_ref[pl.ds(row, CAP), :] = jnp.dot(
                    xg_ref[pl.ds(row, CAP), :], comm_ccw[slot, j],
                    preferred_element_type=jnp.float32,
                ).astype(jnp.bfloat16)

            @pl.when(h < N_DEV - 1)
            def _():
                rdma_cw.wait()
                rdma_ccw.wait()

            @pl.when(h < N_DEV - 2)
            def _():
                pl.semaphore_signal(ack_cw, inc=1, device_id=(lft,),
                                    device_id_type=pl.DeviceIdType.MESH)
                pl.semaphore_signal(ack_ccw, inc=1, device_id=(rgt,),
                                    device_id_type=pl.DeviceIdType.MESH)

        def pair(k, carry):
            do_hop(2 * k, 0)
            do_hop(2 * k + 1, 1)
            return carry

        lax.fori_loop(0, N_DEV // 2, pair, 0)

    return pl.pallas_call(
        body,
        out_shape=jax.ShapeDtypeStruct((n_rows, d_ff), jnp.bfloat16),
        in_specs=[
            pl.BlockSpec(memory_space=pltpu.VMEM),
            pl.BlockSpec(memory_space=pltpu.VMEM),
            pl.BlockSpec(memory_space=pltpu.SMEM),
            pl.BlockSpec(memory_space=pltpu.SMEM),
        ],
        out_specs=pl.BlockSpec(memory_space=pltpu.VMEM),
        scratch_shapes=[
            pltpu.VMEM((2, 2, d_model, d_ff), jnp.bfloat16),
            pltpu.VMEM((2, 2, d_model, d_ff), jnp.bfloat16),
            pltpu.SemaphoreType.DMA((2,)),
            pltpu.SemaphoreType.DMA((2,)),
            pltpu.SemaphoreType.DMA((2,)),
            pltpu.SemaphoreType.DMA((2,)),
            pltpu.SemaphoreType.REGULAR,
            pltpu.SemaphoreType.REGULAR,
        ],
        compiler_params=pltpu.CompilerParams(
            collective_id=0,
            vmem_limit_bytes=100 * 1024 * 1024,
        ),
    )(xg, expert_W, left, right)


def kernel(x, router_W, route_idx, expert_W):
    n_tok, d_model = x.shape
    n_exp = router_W.shape[1]
    n_pairs = 2 * n_tok

    me = lax.axis_index("i")
    cyc = jnp.asarray(_CYCLE)
    c = jnp.asarray(_POS)[me]
    hs = jnp.arange(N_DEV, dtype=jnp.int32)
    src_cw = cyc[(c - hs) % N_DEV]
    src_ccw = cyc[(c + hs) % N_DEV]
    left = jnp.asarray(_PREV)[me][None]
    right = jnp.asarray(_NEXT)[me][None]

    j2 = jnp.arange(2, dtype=jnp.int32)
    e_cw = (4 * src_cw[:, None] + j2).reshape(-1)
    e_ccw = (4 * src_ccw[:, None] + 2 + j2).reshape(-1)
    v = ((2 * hs[:, None] + j2) * CAP).reshape(-1)
    base = (jnp.zeros((n_exp,), jnp.int32)
            .at[e_cw].set(v)
            .at[e_ccw].set(n_exp * CAP // 2 + v))

    scores = x @ router_W
    probs = jax.nn.softmax(scores, axis=-1)
    p = jnp.take_along_axis(probs, route_idx, axis=1)
    g = p / jnp.sum(p, axis=1, keepdims=True)

    pair_e = route_idx.reshape(-1)
    pair_tok = jnp.repeat(jnp.arange(n_tok, dtype=jnp.int32), 2)
    pair_g = g.reshape(-1)
    order = jnp.argsort(pair_e)
    se = pair_e[order]
    st = pair_tok[order]
    sg = pair_g[order]
    idx = jnp.arange(n_pairs, dtype=jnp.int32)
    is_start = jnp.concatenate([jnp.ones((1,), bool), se[1:] != se[:-1]])
    seg_start = lax.cummax(jnp.where(is_start, idx, 0))
    rank = idx - seg_start
    slot = jnp.where(rank < CAP, base[se] + rank, n_exp * CAP)

    xg = jnp.zeros((n_exp * CAP, d_model), jnp.bfloat16)
    xg = xg.at[slot].set((x[st] * sg[:, None]).astype(jnp.bfloat16),
                         mode="drop")

    y = _moe_ring(xg, expert_W.astype(jnp.bfloat16), left, right)

    pair_slot = jnp.zeros((n_pairs,), jnp.int32).at[order].set(slot)
    y0 = jnp.take(y, pair_slot[0::2], axis=0, mode="fill", fill_value=0)
    y1 = jnp.take(y, pair_slot[1::2], axis=0, mode="fill", fill_value=0)
    return y0.astype(jnp.float32) + y1.astype(jnp.float32)
